# baseline (speedup 1.0000x reference)
import functools

import jax
import jax.numpy as jnp
from jax import lax
from jax.experimental import pallas as pl
from jax.experimental.pallas import tpu as pltpu
from jax.experimental.pallas import tpu_sc as plsc

N = 50000
K = 27
C = 128
NW = 32
BC = 112
CHUNKS = 14
NP = NW * BC * CHUNKS
BT = 512
NT = NP // BT
ZROW = N
EPS = 1e-5


def _gemm_body(x_ref, w_ref, o_ref):
    o_ref[...] = jnp.dot(x_ref[...], w_ref[0],
                         preferred_element_type=jnp.float32)


def _gemm_bn_body(y_ref, s_ref, g_ref, b_ref, w_ref, o_ref):
    mean = s_ref[0:1, :] * (1.0 / N)
    var = s_ref[1:2, :] * (1.0 / N) - mean * mean
    inv = lax.rsqrt(var + EPS)
    x = (y_ref[...] - mean) * (inv * g_ref[0:1, :]) + b_ref[0:1, :]
    x = jnp.maximum(x, 0.0)
    t = pl.program_id(0)
    rows = t * BT + lax.broadcasted_iota(jnp.int32, (BT, 1), 0)
    x = jnp.where(rows < N, x, 0.0)
    o_ref[...] = jnp.dot(x, w_ref[0], preferred_element_type=jnp.float32)


def _stats_body(y_ref, o_ref):
    @pl.when(pl.program_id(0) == 0)
    def _init():
        o_ref[...] = jnp.zeros_like(o_ref)

    y = y_ref[...]
    o_ref[0:1, :] += jnp.sum(y, axis=0, keepdims=True)
    o_ref[1:2, :] += jnp.sum(y * y, axis=0, keepdims=True)


def _bn_relu_body(y_ref, s_ref, g_ref, b_ref, o_ref):
    mean = s_ref[0:1, :] * (1.0 / N)
    var = s_ref[1:2, :] * (1.0 / N) - mean * mean
    inv = lax.rsqrt(var + EPS)
    x = (y_ref[...] - mean) * (inv * g_ref[0:1, :]) + b_ref[0:1, :]
    o_ref[...] = jnp.maximum(x, 0.0)


def _gemm_all_k(x, w):
    return pl.pallas_call(
        _gemm_body,
        grid=(NT, K),
        in_specs=[pl.BlockSpec((BT, C), lambda t, k: (t, 0)),
                  pl.BlockSpec((1, C, C), lambda t, k: (k, 0, 0))],
        out_specs=pl.BlockSpec((BT, C), lambda t, k: (k * NT + t, 0)),
        out_shape=jax.ShapeDtypeStruct((K * NP, C), jnp.float32),
    )(x, w)


def _gemm_bn_all_k(y, s, g, b, w):
    return pl.pallas_call(
        _gemm_bn_body,
        grid=(NT, K),
        in_specs=[pl.BlockSpec((BT, C), lambda t, k: (t, 0)),
                  pl.BlockSpec((8, C), lambda t, k: (0, 0)),
                  pl.BlockSpec((1, C), lambda t, k: (0, 0)),
                  pl.BlockSpec((1, C), lambda t, k: (0, 0)),
                  pl.BlockSpec((1, C, C), lambda t, k: (k, 0, 0))],
        out_specs=pl.BlockSpec((BT, C), lambda t, k: (k * NT + t, 0)),
        out_shape=jax.ShapeDtypeStruct((K * NP, C), jnp.float32),
    )(y, s, g, b, w)


def _stats(y):
    return pl.pallas_call(
        _stats_body,
        grid=(NT,),
        in_specs=[pl.BlockSpec((BT, C), lambda t: (t, 0))],
        out_specs=pl.BlockSpec((8, C), lambda t: (0, 0)),
        out_shape=jax.ShapeDtypeStruct((8, C), jnp.float32),
    )(y)


def _bn_relu_final(y, s, g, b):
    return pl.pallas_call(
        _bn_relu_body,
        grid=(25,),
        in_specs=[pl.BlockSpec((2000, C), lambda t: (t, 0)),
                  pl.BlockSpec((8, C), lambda t: (0, 0)),
                  pl.BlockSpec((1, C), lambda t: (0, 0)),
                  pl.BlockSpec((1, C), lambda t: (0, 0))],
        out_specs=pl.BlockSpec((2000, C), lambda t: (t, 0)),
        out_shape=jax.ShapeDtypeStruct((N, C), jnp.float32),
    )(y, s, g, b)


NSET = 3
NROUND = K // NSET


def _reduce_round(bufs_v, acc_v, b0, overwrite):
    cpb = C // 16

    @plsc.parallel_loop(0, BC * cpb, unroll=4)
    def _(p):
        r = p // cpb
        c = (p % cpb) * 16
        s = bufs_v[b0, r, pl.ds(c, 16)]
        for j in range(1, NSET):
            s = s + bufs_v[b0 + j, r, pl.ds(c, 16)]
        if overwrite:
            acc_v[r, pl.ds(c, 16)] = s
        else:
            plsc.addupdate(acc_v.at[r, pl.ds(c, 16)], s)


def _sc_gather_body(z_hbm, idx_hbm, y_hbm, idx_v, bufs_v, acc_v, semg):
    wid = lax.axis_index("s") * 2 + lax.axis_index("c")
    base = wid * (BC * CHUNKS)

    def chunk(ci, carry):
        cb = base + ci * BC
        pltpu.sync_copy(idx_hbm.at[wid * CHUNKS + ci], idx_v)
        prev = None
        for r in range(NROUND):
            b0 = (r % 2) * NSET
            cps = [pltpu.async_copy(z_hbm.at[idx_v.at[r * NSET + j]],
                                    bufs_v.at[b0 + j], semg)
                   for j in range(NSET)]
            if prev is not None:
                pb0, pcps = prev
                for cp in pcps:
                    cp.wait()
                _reduce_round(bufs_v, acc_v, pb0, overwrite=(r == 1))
            prev = (b0, cps)
        pb0, pcps = prev
        for cp in pcps:
            cp.wait()
        _reduce_round(bufs_v, acc_v, pb0, overwrite=False)
        pltpu.sync_copy(acc_v, y_hbm.at[pl.ds(cb, BC)])
        return carry

    lax.fori_loop(0, CHUNKS, chunk, 0)


def _sc_gather(z, idx_t):
    fn = pl.kernel(
        _sc_gather_body,
        out_type=jax.ShapeDtypeStruct((NP, C), jnp.float32),
        mesh=plsc.VectorSubcoreMesh(core_axis_name="c", subcore_axis_name="s"),
        scratch_types=[pltpu.VMEM((K, BC), jnp.int32),
                       pltpu.VMEM((2 * NSET, BC, C), jnp.float32),
                       pltpu.VMEM((BC, C), jnp.float32),
                       pltpu.SemaphoreType.DMA],
    )
    return fn(z, idx_t)


def kernel(features, nbr_idx, nbr_mask, W1, gamma1, beta1, W2, gamma2, beta2):
    x = jnp.pad(features, ((0, NP - N), (0, 0)))
    offs = (jnp.arange(K, dtype=jnp.int32) * NP)[None, :]
    fidx = jnp.where(nbr_mask, nbr_idx + offs, ZROW).astype(jnp.int32)
    fidx = jnp.pad(fidx, ((0, NP - N), (0, 0)), constant_values=ZROW)
    idx_t = fidx.reshape(NW * CHUNKS, BC, K).transpose(0, 2, 1)

    g1 = gamma1.reshape(1, C)
    b1 = beta1.reshape(1, C)
    g2 = gamma2.reshape(1, C)
    b2 = beta2.reshape(1, C)

    z1 = _gemm_all_k(x, W1)
    y1 = _sc_gather(z1, idx_t)
    s1 = _stats(y1)
    z2 = _gemm_bn_all_k(y1, s1, g1, b1, W2)
    y2 = _sc_gather(z2, idx_t)
    s2 = _stats(y2)
    return _bn_relu_final(y2, s2, g2, b2)


# speedup vs baseline: 14.4875x; 14.4875x over previous
import functools

import jax
import jax.numpy as jnp
from jax import lax
from jax.experimental import pallas as pl
from jax.experimental.pallas import tpu as pltpu
from jax.experimental.pallas import tpu_sc as plsc

N = 50000
K = 27
C = 128
NW = 32
BC = 112
CHUNKS = 14
NP = NW * BC * CHUNKS
BT = 512
NT = NP // BT
ZROW = N
EPS = 1e-5


def _gemm_body(x_ref, w_ref, o_ref):
    o_ref[...] = jnp.dot(x_ref[...], w_ref[0],
                         preferred_element_type=jnp.float32)


def _gemm_bn_body(y_ref, s_ref, g_ref, b_ref, w_ref, o_ref):
    mean = s_ref[0:1, :] * (1.0 / N)
    var = s_ref[1:2, :] * (1.0 / N) - mean * mean
    inv = lax.rsqrt(var + EPS)
    x = (y_ref[...] - mean) * (inv * g_ref[0:1, :]) + b_ref[0:1, :]
    x = jnp.maximum(x, 0.0)
    t = pl.program_id(0)
    rows = t * BT + lax.broadcasted_iota(jnp.int32, (BT, 1), 0)
    x = jnp.where(rows < N, x, 0.0)
    o_ref[...] = jnp.dot(x, w_ref[0], preferred_element_type=jnp.float32)


def _stats_body(y_ref, o_ref):
    @pl.when(pl.program_id(0) == 0)
    def _init():
        o_ref[...] = jnp.zeros_like(o_ref)

    y = y_ref[...]
    o_ref[0:1, :] += jnp.sum(y, axis=0, keepdims=True)
    o_ref[1:2, :] += jnp.sum(y * y, axis=0, keepdims=True)


def _bn_relu_body(y_ref, s_ref, g_ref, b_ref, o_ref):
    mean = s_ref[0:1, :] * (1.0 / N)
    var = s_ref[1:2, :] * (1.0 / N) - mean * mean
    inv = lax.rsqrt(var + EPS)
    x = (y_ref[...] - mean) * (inv * g_ref[0:1, :]) + b_ref[0:1, :]
    o_ref[...] = jnp.maximum(x, 0.0)


def _gemm_all_k(x, w):
    return pl.pallas_call(
        _gemm_body,
        grid=(NT, K),
        in_specs=[pl.BlockSpec((BT, C), lambda t, k: (t, 0)),
                  pl.BlockSpec((1, C, C), lambda t, k: (k, 0, 0))],
        out_specs=pl.BlockSpec((BT, C), lambda t, k: (k * NT + t, 0)),
        out_shape=jax.ShapeDtypeStruct((K * NP, C), jnp.float32),
    )(x, w)


def _gemm_bn_all_k(y, s, g, b, w):
    return pl.pallas_call(
        _gemm_bn_body,
        grid=(NT, K),
        in_specs=[pl.BlockSpec((BT, C), lambda t, k: (t, 0)),
                  pl.BlockSpec((8, C), lambda t, k: (0, 0)),
                  pl.BlockSpec((1, C), lambda t, k: (0, 0)),
                  pl.BlockSpec((1, C), lambda t, k: (0, 0)),
                  pl.BlockSpec((1, C, C), lambda t, k: (k, 0, 0))],
        out_specs=pl.BlockSpec((BT, C), lambda t, k: (k * NT + t, 0)),
        out_shape=jax.ShapeDtypeStruct((K * NP, C), jnp.float32),
    )(y, s, g, b, w)


def _stats(y):
    return pl.pallas_call(
        _stats_body,
        grid=(NT,),
        in_specs=[pl.BlockSpec((BT, C), lambda t: (t, 0))],
        out_specs=pl.BlockSpec((8, C), lambda t: (0, 0)),
        out_shape=jax.ShapeDtypeStruct((8, C), jnp.float32),
    )(y)


def _bn_relu_final(y, s, g, b):
    return pl.pallas_call(
        _bn_relu_body,
        grid=(25,),
        in_specs=[pl.BlockSpec((2000, C), lambda t: (t, 0)),
                  pl.BlockSpec((8, C), lambda t: (0, 0)),
                  pl.BlockSpec((1, C), lambda t: (0, 0)),
                  pl.BlockSpec((1, C), lambda t: (0, 0))],
        out_specs=pl.BlockSpec((2000, C), lambda t: (t, 0)),
        out_shape=jax.ShapeDtypeStruct((N, C), jnp.float32),
    )(y, s, g, b)


NSET = 3
NROUND = K // NSET


def _reduce_round(bufs_v, acc_v, b0, overwrite):
    cpb = C // 16

    @plsc.parallel_loop(0, BC * cpb, unroll=4)
    def _(p):
        r = p // cpb
        c = (p % cpb) * 16
        s = bufs_v[b0, r, pl.ds(c, 16)]
        for j in range(1, NSET):
            s = s + bufs_v[b0 + j, r, pl.ds(c, 16)]
        if overwrite:
            acc_v[r, pl.ds(c, 16)] = s
        else:
            plsc.addupdate(acc_v.at[r, pl.ds(c, 16)], s)


def _sc_gather_body(z_hbm, idx_hbm, y_hbm, idx_v, bufs_v, acc_v, semg):
    wid = lax.axis_index("s") * 2 + lax.axis_index("c")
    base = wid * (BC * CHUNKS)

    def chunk(ci, carry):
        cb = base + ci * BC
        pltpu.sync_copy(idx_hbm.at[wid * CHUNKS + ci], idx_v)
        prev = None
        for r in range(NROUND):
            b0 = (r % 2) * NSET
            cps = [pltpu.async_copy(z_hbm.at[idx_v.at[r * NSET + j]],
                                    bufs_v.at[b0 + j], semg)
                   for j in range(NSET)]
            if prev is not None:
                pb0, pcps = prev
                for cp in pcps:
                    cp.wait()
                _reduce_round(bufs_v, acc_v, pb0, overwrite=(r == 1))
            prev = (b0, cps)
        pb0, pcps = prev
        for cp in pcps:
            cp.wait()
        _reduce_round(bufs_v, acc_v, pb0, overwrite=False)
        pltpu.sync_copy(acc_v, y_hbm.at[pl.ds(cb, BC)])
        return carry

    lax.fori_loop(0, CHUNKS, chunk, 0)


def _sc_gather(z, idx_t):
    fn = pl.kernel(
        _sc_gather_body,
        out_type=jax.ShapeDtypeStruct((NP, C), jnp.float32),
        mesh=plsc.VectorSubcoreMesh(core_axis_name="c", subcore_axis_name="s"),
        scratch_types=[pltpu.VMEM((K, BC), jnp.int32),
                       pltpu.VMEM((2 * NSET, BC, C), jnp.float32),
                       pltpu.VMEM((BC, C), jnp.float32),
                       pltpu.SemaphoreType.DMA],
    )
    return fn(z, idx_t)


def kernel(features, nbr_idx, nbr_mask, W1, gamma1, beta1, W2, gamma2, beta2):
    x = jnp.pad(features, ((0, NP - N), (0, 0)))
    offs = (jnp.arange(K, dtype=jnp.int32) * NP)[None, :]
    pad_rows = (N + (jnp.arange(N, dtype=jnp.int32) % (NP - N)))[:, None]
    fidx = jnp.where(nbr_mask, nbr_idx + offs, pad_rows + offs).astype(jnp.int32)
    self_pad = jnp.arange(N, NP, dtype=jnp.int32)[:, None] + offs
    fidx = jnp.concatenate([fidx, self_pad], axis=0)
    idx_t = fidx.reshape(NW * CHUNKS, BC, K).transpose(0, 2, 1)

    g1 = gamma1.reshape(1, C)
    b1 = beta1.reshape(1, C)
    g2 = gamma2.reshape(1, C)
    b2 = beta2.reshape(1, C)

    z1 = _gemm_all_k(x, W1)
    y1 = _sc_gather(z1, idx_t)
    s1 = _stats(y1)
    z2 = _gemm_bn_all_k(y1, s1, g1, b1, W2)
    y2 = _sc_gather(z2, idx_t)
    s2 = _stats(y2)
    return _bn_relu_final(y2, s2, g2, b2)


# speedup vs baseline: 43.7331x; 3.0187x over previous
import jax
import jax.numpy as jnp
from jax import lax
from jax.experimental import pallas as pl
from jax.experimental.pallas import tpu as pltpu
from jax.experimental.pallas import tpu_sc as plsc

N = 50000
K = 27
KA = 12
C = 128
NW = 32
BC = 112
CHUNKS = 14
NP = NW * BC * CHUNKS
BT = 12544
NT = NP // BT
EPS = 1e-5


def _gemm_body(x_ref, w_ref, o_ref):
    o_ref[...] = jnp.dot(x_ref[...].astype(jnp.bfloat16),
                         w_ref[0].astype(jnp.bfloat16),
                         preferred_element_type=jnp.float32)


def _gemm_bn_body(ya_ref, yb_ref, s_ref, g_ref, b_ref, w_ref, o_ref):
    mean = s_ref[0:1, :] * (1.0 / N)
    var = s_ref[1:2, :] * (1.0 / N) - mean * mean
    inv = lax.rsqrt(var + EPS)
    y = ya_ref[...] + yb_ref[...]
    x = (y - mean) * (inv * g_ref[0:1, :]) + b_ref[0:1, :]
    x = jnp.maximum(x, 0.0)
    t = pl.program_id(0)
    rows = t * BT + lax.broadcasted_iota(jnp.int32, (BT, 1), 0)
    x = jnp.where(rows < N, x, 0.0)
    o_ref[...] = jnp.dot(x.astype(jnp.bfloat16),
                         w_ref[0].astype(jnp.bfloat16),
                         preferred_element_type=jnp.float32)


def _stats_body(ya_ref, yb_ref, o_ref):
    @pl.when(pl.program_id(0) == 0)
    def _init():
        o_ref[...] = jnp.zeros_like(o_ref)

    y = ya_ref[...] + yb_ref[...]
    o_ref[0:1, :] += jnp.sum(y, axis=0, keepdims=True)
    o_ref[1:2, :] += jnp.sum(y * y, axis=0, keepdims=True)


def _bn_relu_body(ya_ref, yb_ref, s_ref, g_ref, b_ref, o_ref):
    mean = s_ref[0:1, :] * (1.0 / N)
    var = s_ref[1:2, :] * (1.0 / N) - mean * mean
    inv = lax.rsqrt(var + EPS)
    y = ya_ref[...] + yb_ref[...]
    x = (y - mean) * (inv * g_ref[0:1, :]) + b_ref[0:1, :]
    o_ref[...] = jnp.maximum(x, 0.0)


def _gemm_group(x, w):
    kg = w.shape[0]
    return pl.pallas_call(
        _gemm_body,
        grid=(NT, kg),
        in_specs=[pl.BlockSpec((BT, C), lambda t, k: (t, 0)),
                  pl.BlockSpec((1, C, C), lambda t, k: (k, 0, 0))],
        out_specs=pl.BlockSpec((BT, C), lambda t, k: (k * NT + t, 0)),
        out_shape=jax.ShapeDtypeStruct((kg * NP, C), jnp.float32),
    )(x, w)


def _gemm_bn_group(ya, yb, s, g, b, w):
    kg = w.shape[0]
    return pl.pallas_call(
        _gemm_bn_body,
        grid=(NT, kg),
        in_specs=[pl.BlockSpec((BT, C), lambda t, k: (t, 0)),
                  pl.BlockSpec((BT, C), lambda t, k: (t, 0)),
                  pl.BlockSpec((8, C), lambda t, k: (0, 0)),
                  pl.BlockSpec((1, C), lambda t, k: (0, 0)),
                  pl.BlockSpec((1, C), lambda t, k: (0, 0)),
                  pl.BlockSpec((1, C, C), lambda t, k: (k, 0, 0))],
        out_specs=pl.BlockSpec((BT, C), lambda t, k: (k * NT + t, 0)),
        out_shape=jax.ShapeDtypeStruct((kg * NP, C), jnp.float32),
    )(ya, yb, s, g, b, w)


def _stats(ya, yb):
    return pl.pallas_call(
        _stats_body,
        grid=(NT,),
        in_specs=[pl.BlockSpec((BT, C), lambda t: (t, 0)),
                  pl.BlockSpec((BT, C), lambda t: (t, 0))],
        out_specs=pl.BlockSpec((8, C), lambda t: (0, 0)),
        out_shape=jax.ShapeDtypeStruct((8, C), jnp.float32),
    )(ya, yb)


def _bn_relu_final(ya, yb, s, g, b):
    return pl.pallas_call(
        _bn_relu_body,
        grid=(25,),
        in_specs=[pl.BlockSpec((2000, C), lambda t: (t, 0)),
                  pl.BlockSpec((2000, C), lambda t: (t, 0)),
                  pl.BlockSpec((8, C), lambda t: (0, 0)),
                  pl.BlockSpec((1, C), lambda t: (0, 0)),
                  pl.BlockSpec((1, C), lambda t: (0, 0))],
        out_specs=pl.BlockSpec((2000, C), lambda t: (t, 0)),
        out_shape=jax.ShapeDtypeStruct((N, C), jnp.float32),
    )(ya, yb, s, g, b)


NSET = 3


def _reduce_round(bufs_v, acc_v, b0, overwrite):
    cpb = C // 16

    @plsc.parallel_loop(0, BC * cpb, unroll=4)
    def _(p):
        r = p // cpb
        c = (p % cpb) * 16
        s = bufs_v[b0, r, pl.ds(c, 16)]
        for j in range(1, NSET):
            s = s + bufs_v[b0 + j, r, pl.ds(c, 16)]
        if overwrite:
            acc_v[r, pl.ds(c, 16)] = s
        else:
            plsc.addupdate(acc_v.at[r, pl.ds(c, 16)], s)


def _make_sc_body(kg):
    nround = kg // NSET

    def body(z_hbm, idx_hbm, y_hbm, idx_v, bufs_v, acc_v, semg):
        wid = lax.axis_index("s") * 2 + lax.axis_index("c")
        base = wid * (BC * CHUNKS)

        def chunk(ci, carry):
            cb = base + ci * BC
            pltpu.sync_copy(idx_hbm.at[wid * CHUNKS + ci], idx_v)
            prev = None
            for r in range(nround):
                b0 = (r % 2) * NSET
                cps = [pltpu.async_copy(z_hbm.at[idx_v.at[r * NSET + j]],
                                        bufs_v.at[b0 + j], semg)
                       for j in range(NSET)]
                if prev is not None:
                    pb0, pcps = prev
                    for cp in pcps:
                        cp.wait()
                    _reduce_round(bufs_v, acc_v, pb0, overwrite=(r == 1))
                prev = (b0, cps)
            pb0, pcps = prev
            for cp in pcps:
                cp.wait()
            _reduce_round(bufs_v, acc_v, pb0, overwrite=False)
            pltpu.sync_copy(acc_v, y_hbm.at[pl.ds(cb, BC)])
            return carry

        lax.fori_loop(0, CHUNKS, chunk, 0)

    return body


def _sc_gather(z, idx_t):
    kg = idx_t.shape[1]
    fn = pl.kernel(
        _make_sc_body(kg),
        out_type=jax.ShapeDtypeStruct((NP, C), jnp.float32),
        mesh=plsc.VectorSubcoreMesh(core_axis_name="c", subcore_axis_name="s"),
        scratch_types=[pltpu.VMEM((kg, BC), jnp.int32),
                       pltpu.VMEM((2 * NSET, BC, C), jnp.float32),
                       pltpu.VMEM((BC, C), jnp.float32),
                       pltpu.SemaphoreType.DMA],
    )
    return fn(z, idx_t)


def kernel(features, nbr_idx, nbr_mask, W1, gamma1, beta1, W2, gamma2, beta2):
    x = jnp.pad(features, ((0, NP - N), (0, 0)))
    offs = (jnp.arange(K, dtype=jnp.int32) * NP)[None, :]
    pad_rows = (N + (jnp.arange(N, dtype=jnp.int32) % (NP - N)))[:, None]
    fidx = jnp.where(nbr_mask, nbr_idx + offs, pad_rows + offs).astype(jnp.int32)
    self_pad = jnp.arange(N, NP, dtype=jnp.int32)[:, None] + offs
    fidx = jnp.concatenate([fidx, self_pad], axis=0)
    idx_t = fidx.reshape(NW * CHUNKS, BC, K).transpose(0, 2, 1)
    idx_a = idx_t[:, :KA]
    idx_b = idx_t[:, KA:] - KA * NP

    g1 = gamma1.reshape(1, C)
    b1 = beta1.reshape(1, C)
    g2 = gamma2.reshape(1, C)
    b2 = beta2.reshape(1, C)

    z1a = _gemm_group(x, W1[:KA])
    y1a = _sc_gather(z1a, idx_a)
    z1b = _gemm_group(x, W1[KA:])
    y1b = _sc_gather(z1b, idx_b)
    s1 = _stats(y1a, y1b)
    z2a = _gemm_bn_group(y1a, y1b, s1, g1, b1, W2[:KA])
    y2a = _sc_gather(z2a, idx_a)
    z2b = _gemm_bn_group(y1a, y1b, s1, g1, b1, W2[KA:])
    y2b = _sc_gather(z2b, idx_b)
    s2 = _stats(y2a, y2b)
    return _bn_relu_final(y2a, y2b, s2, g2, b2)


# speedup vs baseline: 47.4135x; 1.0842x over previous
import functools

import jax
import jax.numpy as jnp
from jax import lax
from jax.experimental import pallas as pl
from jax.experimental.pallas import tpu as pltpu
from jax.experimental.pallas import tpu_sc as plsc

N = 50000
K = 27
C = 128
NW = 32
BC = 112
CHUNKS = 14
NP = NW * BC * CHUNKS
BT = 12544
NT = NP // BT
ZROW = N
EPS = 1e-5


def _gemm_body(x_ref, w_ref, o_ref):
    o_ref[...] = jnp.dot(x_ref[...].astype(jnp.bfloat16),
                         w_ref[0].astype(jnp.bfloat16),
                         preferred_element_type=jnp.float32)


def _gemm_bn_body(y_ref, s_ref, g_ref, b_ref, w_ref, o_ref):
    mean = s_ref[0:1, :] * (1.0 / N)
    var = s_ref[1:2, :] * (1.0 / N) - mean * mean
    inv = lax.rsqrt(var + EPS)
    x = (y_ref[...] - mean) * (inv * g_ref[0:1, :]) + b_ref[0:1, :]
    x = jnp.maximum(x, 0.0)
    t = pl.program_id(0)
    rows = t * BT + lax.broadcasted_iota(jnp.int32, (BT, 1), 0)
    x = jnp.where(rows < N, x, 0.0)
    o_ref[...] = jnp.dot(x.astype(jnp.bfloat16),
                         w_ref[0].astype(jnp.bfloat16),
                         preferred_element_type=jnp.float32)


def _stats_body(y_ref, o_ref):
    @pl.when(pl.program_id(0) == 0)
    def _init():
        o_ref[...] = jnp.zeros_like(o_ref)

    y = y_ref[...]
    o_ref[0:1, :] += jnp.sum(y, axis=0, keepdims=True)
    o_ref[1:2, :] += jnp.sum(y * y, axis=0, keepdims=True)


def _bn_relu_body(y_ref, s_ref, g_ref, b_ref, o_ref):
    mean = s_ref[0:1, :] * (1.0 / N)
    var = s_ref[1:2, :] * (1.0 / N) - mean * mean
    inv = lax.rsqrt(var + EPS)
    x = (y_ref[...] - mean) * (inv * g_ref[0:1, :]) + b_ref[0:1, :]
    o_ref[...] = jnp.maximum(x, 0.0)


def _gemm_all_k(x, w):
    return pl.pallas_call(
        _gemm_body,
        grid=(NT, K),
        in_specs=[pl.BlockSpec((BT, C), lambda t, k: (t, 0)),
                  pl.BlockSpec((1, C, C), lambda t, k: (k, 0, 0))],
        out_specs=pl.BlockSpec((BT, C), lambda t, k: (k * NT + t, 0)),
        out_shape=jax.ShapeDtypeStruct((K * NP, C), jnp.float32),
    )(x, w)


def _gemm_bn_all_k(y, s, g, b, w):
    return pl.pallas_call(
        _gemm_bn_body,
        grid=(NT, K),
        in_specs=[pl.BlockSpec((BT, C), lambda t, k: (t, 0)),
                  pl.BlockSpec((8, C), lambda t, k: (0, 0)),
                  pl.BlockSpec((1, C), lambda t, k: (0, 0)),
                  pl.BlockSpec((1, C), lambda t, k: (0, 0)),
                  pl.BlockSpec((1, C, C), lambda t, k: (k, 0, 0))],
        out_specs=pl.BlockSpec((BT, C), lambda t, k: (k * NT + t, 0)),
        out_shape=jax.ShapeDtypeStruct((K * NP, C), jnp.float32),
    )(y, s, g, b, w)


def _stats(y):
    return pl.pallas_call(
        _stats_body,
        grid=(NT,),
        in_specs=[pl.BlockSpec((BT, C), lambda t: (t, 0))],
        out_specs=pl.BlockSpec((8, C), lambda t: (0, 0)),
        out_shape=jax.ShapeDtypeStruct((8, C), jnp.float32),
    )(y)


def _bn_relu_final(y, s, g, b):
    return pl.pallas_call(
        _bn_relu_body,
        grid=(25,),
        in_specs=[pl.BlockSpec((2000, C), lambda t: (t, 0)),
                  pl.BlockSpec((8, C), lambda t: (0, 0)),
                  pl.BlockSpec((1, C), lambda t: (0, 0)),
                  pl.BlockSpec((1, C), lambda t: (0, 0))],
        out_specs=pl.BlockSpec((2000, C), lambda t: (t, 0)),
        out_shape=jax.ShapeDtypeStruct((N, C), jnp.float32),
    )(y, s, g, b)


NSET = 3
NROUND = K // NSET


def _reduce_round(bufs_v, acc_v, b0, overwrite):
    cpb = C // 16

    @plsc.parallel_loop(0, BC * cpb, unroll=4)
    def _(p):
        r = p // cpb
        c = (p % cpb) * 16
        s = bufs_v[b0, r, pl.ds(c, 16)]
        for j in range(1, NSET):
            s = s + bufs_v[b0 + j, r, pl.ds(c, 16)]
        if overwrite:
            acc_v[r, pl.ds(c, 16)] = s
        else:
            plsc.addupdate(acc_v.at[r, pl.ds(c, 16)], s)


def _sc_gather_body(z_hbm, idx_hbm, y_hbm, idx_v, bufs_v, acc_v, semg):
    wid = lax.axis_index("s") * 2 + lax.axis_index("c")
    base = wid * (BC * CHUNKS)

    def chunk(ci, carry):
        cb = base + ci * BC
        pltpu.sync_copy(idx_hbm.at[wid * CHUNKS + ci], idx_v)
        prev = None
        for r in range(NROUND):
            b0 = (r % 2) * NSET
            cps = [pltpu.async_copy(z_hbm.at[idx_v.at[r * NSET + j]],
                                    bufs_v.at[b0 + j], semg)
                   for j in range(NSET)]
            if prev is not None:
                pb0, pcps = prev
                for cp in pcps:
                    cp.wait()
                _reduce_round(bufs_v, acc_v, pb0, overwrite=(r == 1))
            prev = (b0, cps)
        pb0, pcps = prev
        for cp in pcps:
            cp.wait()
        _reduce_round(bufs_v, acc_v, pb0, overwrite=False)
        pltpu.sync_copy(acc_v, y_hbm.at[pl.ds(cb, BC)])
        return carry

    lax.fori_loop(0, CHUNKS, chunk, 0)


def _sc_gather(z, idx_t):
    fn = pl.kernel(
        _sc_gather_body,
        out_type=jax.ShapeDtypeStruct((NP, C), jnp.float32),
        mesh=plsc.VectorSubcoreMesh(core_axis_name="c", subcore_axis_name="s"),
        scratch_types=[pltpu.VMEM((K, BC), jnp.int32),
                       pltpu.VMEM((2 * NSET, BC, C), jnp.float32),
                       pltpu.VMEM((BC, C), jnp.float32),
                       pltpu.SemaphoreType.DMA],
    )
    return fn(z, idx_t)


def kernel(features, nbr_idx, nbr_mask, W1, gamma1, beta1, W2, gamma2, beta2):
    x = jnp.pad(features, ((0, NP - N), (0, 0)))
    offs = (jnp.arange(K, dtype=jnp.int32) * NP)[None, :]
    pad_rows = (N + (jnp.arange(N, dtype=jnp.int32) % (NP - N)))[:, None]
    fidx = jnp.where(nbr_mask, nbr_idx + offs, pad_rows + offs).astype(jnp.int32)
    self_pad = jnp.arange(N, NP, dtype=jnp.int32)[:, None] + offs
    fidx = jnp.concatenate([fidx, self_pad], axis=0)
    idx_t = fidx.reshape(NW * CHUNKS, BC, K).transpose(0, 2, 1)

    g1 = gamma1.reshape(1, C)
    b1 = beta1.reshape(1, C)
    g2 = gamma2.reshape(1, C)
    b2 = beta2.reshape(1, C)

    z1 = _gemm_all_k(x, W1)
    y1 = _sc_gather(z1, idx_t)
    s1 = _stats(y1)
    z2 = _gemm_bn_all_k(y1, s1, g1, b1, W2)
    y2 = _sc_gather(z2, idx_t)
    s2 = _stats(y2)
    return _bn_relu_final(y2, s2, g2, b2)
